# Initial kernel scaffold; baseline (speedup 1.0000x reference)
import functools

import jax
import jax.numpy as jnp
from jax import lax
from jax.experimental import pallas as pl
from jax.experimental.pallas import tpu as pltpu
from jax.experimental.pallas import tpu_sc as plsc

B = 512
NP = 40
EP = 80
D = 128
NCOUT = 3
E = B * EP

NW = 32
GPW = B // NW
EPW = GPW * EP
APW = GPW * NP * NP

MOL_G = 16
SOLV_G = 64


@functools.partial(
    pl.kernel,
    out_type=jax.ShapeDtypeStruct((3, NW, APW), jnp.float32),
    mesh=plsc.VectorSubcoreMesh(core_axis_name="c", subcore_axis_name="s"),
    scratch_types=[
        pltpu.VMEM((EPW,), jnp.int32),
        pltpu.VMEM((EPW,), jnp.int32),
        pltpu.VMEM((APW,), jnp.float32),
    ],
)
def _sc_adj(ei_ref, zeros_ref, out_ref, src_v, dst_v, acc_v):
    wid = lax.axis_index("s") * 2 + lax.axis_index("c")
    base_e = wid * EPW
    ones = jnp.ones((16,), jnp.float32)
    for m in range(3):
        pltpu.sync_copy(ei_ref.at[m, 0, pl.ds(base_e, EPW)], src_v)
        pltpu.sync_copy(ei_ref.at[m, 1, pl.ds(base_e, EPW)], dst_v)
        pltpu.sync_copy(zeros_ref, acc_v)
        for i in range(EPW // 16):
            g_loc = i // 5
            s = src_v[pl.ds(i * 16, 16)]
            d = dst_v[pl.ds(i * 16, 16)]
            nb = (wid * GPW + g_loc) * NP
            flat = (d - nb) * NP + (s - nb) + g_loc * (NP * NP)
            plsc.addupdate_scatter(acc_v, [flat], ones)
        pltpu.sync_copy(acc_v, out_ref.at[m, wid])


def _mol_body(sol_ref, A_ref, h1_ref, h2_ref, h3_ref,
              W1_ref, b1_ref, W2_ref, b2_ref, out_ref):
    i = pl.program_id(0)
    W1 = W1_ref[...]
    b1 = b1_ref[...]
    W2 = W2_ref[...]
    b2 = b2_ref[...]
    for m, h_ref in enumerate((h1_ref, h2_ref, h3_ref)):
        A = A_ref[m]
        din = jnp.maximum(jnp.sum(A, axis=2), 1.0)
        dout = jnp.maximum(jnp.sum(A, axis=1), 1.0)
        Ah = A * lax.rsqrt(din)[:, :, None] * lax.rsqrt(dout)[:, None, :]
        t = h_ref[...]
        for W, bb in ((W1, b1), (W2, b2)):
            agg = lax.dot_general(
                Ah, t, (((2,), (1,)), ((0,), (0,))),
                preferred_element_type=jnp.float32)
            tw = lax.dot_general(
                agg.reshape(MOL_G * NP, D), W, (((1,), (0,)), ((), ())),
                preferred_element_type=jnp.float32)
            t = jnp.maximum(tw.reshape(MOL_G, NP, D) + bb, 0.0)
        sol = sol_ref[m, pl.ds(i * MOL_G, MOL_G)]
        out_ref[m] = jnp.mean(t, axis=1) * sol[:, None]


def _mol_call(sol, A, h1r, h2r, h3r, W1, b1, W2, b2):
    g = B // MOL_G
    hspec = pl.BlockSpec((MOL_G, NP, D), lambda i: (i, 0, 0))
    full2 = pl.BlockSpec((1, D), lambda i: (0, 0))
    return pl.pallas_call(
        _mol_body,
        grid=(g,),
        in_specs=[
            pl.BlockSpec((3, B), lambda i: (0, 0)),
            pl.BlockSpec((3, MOL_G, NP, NP), lambda i: (0, i, 0, 0)),
            hspec, hspec, hspec,
            pl.BlockSpec((D, D), lambda i: (0, 0)), full2,
            pl.BlockSpec((D, D), lambda i: (0, 0)), full2,
        ],
        out_specs=pl.BlockSpec((3, MOL_G, D), lambda i: (0, i, 0)),
        out_shape=jax.ShapeDtypeStruct((3, B, D), jnp.float32),
    )(sol, A, h1r, h2r, h3r, W1, b1, W2, b2)


def _md(x, w):
    t, g, k = x.shape
    r = lax.dot_general(x.reshape(t * g, k), w, (((1,), (0,)), ((), ())),
                        preferred_element_type=jnp.float32)
    return r.reshape(t, g, w.shape[1])


def _solv_body(ef_ref, hg_ref, Wp_ref, bp_ref, We1_ref, be1_ref,
               W2p_ref, be2r_ref, bnn_ref, WihT_ref, WhhT_ref,
               bih_ref, bhh_ref, C1_ref, C1b_ref, C2_ref, C2b_ref,
               C3p_ref, C3b_ref, out_ref):
    hg = hg_ref[...]
    node = jnp.maximum(_md(hg, Wp_ref[...]) + bp_ref[...], 0.0)
    U = _md(node, W2p_ref[...]).reshape(3, SOLV_G, 32, D)
    v = _md(node, be2r_ref[...])
    ef = ef_ref[...]
    We1 = We1_ref[...]
    be1 = be1_ref[...]

    def msg(s, j):
        a = jnp.maximum(ef[:, j][:, None] * We1 + be1, 0.0)
        return jnp.sum(a[:, :, None] * U[s], axis=1) + v[s]

    bnn = bnn_ref[...]
    agg0 = msg(1, 0) + msg(2, 1) + msg(0, 3) + bnn
    agg1 = msg(0, 0) + msg(2, 2) + msg(1, 4) + bnn
    agg2 = msg(0, 1) + msg(1, 2) + msg(2, 5) + bnn
    mrel = jnp.maximum(jnp.stack([agg0, agg1, agg2]), 0.0)

    gi = _md(mrel, WihT_ref[...]) + bih_ref[...]
    gh = _md(node, WhhT_ref[...]) + bhh_ref[...]
    r = jax.nn.sigmoid(gi[..., :D] + gh[..., :D])
    z = jax.nn.sigmoid(gi[..., D:2 * D] + gh[..., D:2 * D])
    ng = jnp.tanh(gi[..., 2 * D:] + r * gh[..., 2 * D:])
    nod = (1.0 - z) * ng + z * node

    cat = jnp.concatenate([nod[0], nod[1], nod[2]], axis=1)
    o = jnp.maximum(
        lax.dot_general(cat, C1_ref[...], (((1,), (0,)), ((), ())),
                        preferred_element_type=jnp.float32) + C1b_ref[...], 0.0)
    o = jnp.maximum(
        lax.dot_general(o, C2_ref[...], (((1,), (0,)), ((), ())),
                        preferred_element_type=jnp.float32) + C2b_ref[...], 0.0)
    out_ref[...] = lax.dot_general(
        o, C3p_ref[...], (((1,), (0,)), ((), ())),
        preferred_element_type=jnp.float32) + C3b_ref[...]


def _solv_call(ef, hg, Wp, bp, We1, be1, W2p, be2r, bnn,
               WihT, WhhT, bih, bhh, C1, C1b, C2, C2b, C3p, C3b):
    g = B // SOLV_G

    def fixed(*shape):
        n = len(shape)
        return pl.BlockSpec(shape, lambda i, _n=n: (0,) * _n)

    return pl.pallas_call(
        _solv_body,
        grid=(g,),
        in_specs=[
            pl.BlockSpec((SOLV_G, 6), lambda i: (i, 0)),
            pl.BlockSpec((3, SOLV_G, D), lambda i: (0, i, 0)),
            fixed(D, D), fixed(1, D), fixed(1, 32), fixed(1, 32),
            fixed(D, 32 * D), fixed(D, D), fixed(1, D),
            fixed(D, 3 * D), fixed(D, 3 * D), fixed(1, 3 * D), fixed(1, 3 * D),
            fixed(3 * D, D), fixed(1, D), fixed(D, D), fixed(1, D),
            fixed(D, D), fixed(1, D),
        ],
        out_specs=pl.BlockSpec((SOLV_G, D), lambda i: (i, 0)),
        out_shape=jax.ShapeDtypeStruct((B, D), jnp.float32),
    )(ef, hg, Wp, bp, We1, be1, W2p, be2r, bnn,
      WihT, WhhT, bih, bhh, C1, C1b, C2, C2b, C3p, C3b)


def kernel(h1, h2, h3, solv1_x, solv2_x, inter_hb12, inter_hb13, inter_hb23,
           intra_hb1, intra_hb2, intra_hb3, W1, b1, W2, b2, Wp, bp,
           We1, be1, We2, be2, bnn, W_ih, W_hh, b_ih, b_hh,
           C1, C1b, C2, C2b, C3, C3b,
           edge_index1, edge_index2, edge_index3):
    ei = jnp.stack([edge_index1, edge_index2, edge_index3])
    zeros = jnp.zeros((APW,), jnp.float32)
    A = _sc_adj(ei, zeros).reshape(3, B, NP, NP)

    sol = jnp.stack([solv1_x, solv2_x, 1.0 - solv1_x - solv2_x])
    hg = _mol_call(sol, A,
                   h1.reshape(B, NP, D), h2.reshape(B, NP, D),
                   h3.reshape(B, NP, D),
                   W1, b1.reshape(1, D), W2, b2.reshape(1, D))

    ef = jnp.stack([inter_hb12, inter_hb13, inter_hb23,
                    intra_hb1, intra_hb2, intra_hb3], axis=1)
    W2p = We2.reshape(32, D, D).transpose(1, 0, 2).reshape(D, 32 * D)
    be2r = be2.reshape(D, D)
    C3p = jnp.pad(C3, ((0, 0), (0, D - NCOUT)))
    C3bp = jnp.pad(C3b, (0, D - NCOUT))

    out = _solv_call(ef, hg, Wp, bp.reshape(1, D), We1, be1.reshape(1, 32),
                     W2p, be2r, bnn.reshape(1, D),
                     W_ih.T, W_hh.T, b_ih.reshape(1, 3 * D),
                     b_hh.reshape(1, 3 * D),
                     C1, C1b.reshape(1, D), C2, C2b.reshape(1, D),
                     C3p, C3bp.reshape(1, D))
    return out[:, :NCOUT]


# speedup vs baseline: 11.9540x; 11.9540x over previous
import functools

import jax
import jax.numpy as jnp
from jax import lax
from jax.experimental import pallas as pl
from jax.experimental.pallas import tpu as pltpu
from jax.experimental.pallas import tpu_sc as plsc

B = 512
NP = 40
EP = 80
D = 128
NCOUT = 3
E = B * EP

NW = 32
GPW = B // NW
EPW = GPW * EP
APW = GPW * NP * NP

MOL_G = 16
SOLV_G = 64


@functools.partial(
    pl.kernel,
    out_type=jax.ShapeDtypeStruct((3, NW, APW), jnp.float32),
    mesh=plsc.VectorSubcoreMesh(core_axis_name="c", subcore_axis_name="s",
                                num_cores=2, num_subcores=16),
    compiler_params=pltpu.CompilerParams(needs_layout_passes=False),
    scratch_types=[
        pltpu.VMEM((EPW,), jnp.int32),
        pltpu.VMEM((EPW,), jnp.int32),
        pltpu.VMEM((APW,), jnp.float32),
    ],
)
def _sc_adj(ei_ref, zeros_ref, out_ref, src_v, dst_v, acc_v):
    wid = lax.axis_index("s") * 2 + lax.axis_index("c")
    base_e = wid * EPW
    ones = jnp.ones((16,), jnp.float32)
    for m in range(3):
        pltpu.sync_copy(ei_ref.at[m, 0, pl.ds(base_e, EPW)], src_v)
        pltpu.sync_copy(ei_ref.at[m, 1, pl.ds(base_e, EPW)], dst_v)
        pltpu.sync_copy(zeros_ref, acc_v)
        for i in range(EPW // 16):
            g_loc = i // 5
            s = src_v[pl.ds(i * 16, 16)]
            d = dst_v[pl.ds(i * 16, 16)]
            nb = (wid * GPW + g_loc) * NP
            flat = (d - nb) * NP + (s - nb) + g_loc * (NP * NP)
            plsc.addupdate_scatter(acc_v, [flat], ones)
        pltpu.sync_copy(acc_v, out_ref.at[m, wid])


def _mol_body(sol_ref, A_ref, h1_ref, h2_ref, h3_ref,
              W1_ref, b1_ref, W2_ref, b2_ref, out_ref):
    W1 = W1_ref[...]
    b1 = b1_ref[...]
    W2 = W2_ref[...]
    b2 = b2_ref[...]
    for m, h_ref in enumerate((h1_ref, h2_ref, h3_ref)):
        A = A_ref[m]
        din = jnp.maximum(jnp.sum(A, axis=2), 1.0)
        dout = jnp.maximum(jnp.sum(A, axis=1), 1.0)
        Ah = A * lax.rsqrt(din)[:, :, None] * lax.rsqrt(dout)[:, None, :]
        t = h_ref[...]
        for W, bb in ((W1, b1), (W2, b2)):
            agg = lax.dot_general(
                Ah, t, (((2,), (1,)), ((0,), (0,))),
                preferred_element_type=jnp.float32)
            tw = lax.dot_general(
                agg.reshape(MOL_G * NP, D), W, (((1,), (0,)), ((), ())),
                preferred_element_type=jnp.float32)
            t = jnp.maximum(tw.reshape(MOL_G, NP, D) + bb, 0.0)
        sol = sol_ref[...][:, m]
        out_ref[m] = jnp.mean(t, axis=1) * sol[:, None]


def _mol_call(sol, A, h1r, h2r, h3r, W1, b1, W2, b2):
    g = B // MOL_G
    hspec = pl.BlockSpec((MOL_G, NP, D), lambda i: (i, 0, 0))
    full2 = pl.BlockSpec((1, D), lambda i: (0, 0))
    return pl.pallas_call(
        _mol_body,
        grid=(g,),
        in_specs=[
            pl.BlockSpec((MOL_G, 3), lambda i: (i, 0)),
            pl.BlockSpec((3, MOL_G, NP, NP), lambda i: (0, i, 0, 0)),
            hspec, hspec, hspec,
            pl.BlockSpec((D, D), lambda i: (0, 0)), full2,
            pl.BlockSpec((D, D), lambda i: (0, 0)), full2,
        ],
        out_specs=pl.BlockSpec((3, MOL_G, D), lambda i: (0, i, 0)),
        out_shape=jax.ShapeDtypeStruct((3, B, D), jnp.float32),
    )(sol, A, h1r, h2r, h3r, W1, b1, W2, b2)


def _md(x, w):
    t, g, k = x.shape
    r = lax.dot_general(x.reshape(t * g, k), w, (((1,), (0,)), ((), ())),
                        preferred_element_type=jnp.float32)
    return r.reshape(t, g, w.shape[1])


def _solv_body(ef_ref, hg_ref, Wp_ref, bp_ref, We1_ref, be1_ref,
               W2p_ref, be2r_ref, bnn_ref, WihT_ref, WhhT_ref,
               bih_ref, bhh_ref, C1_ref, C1b_ref, C2_ref, C2b_ref,
               C3p_ref, C3b_ref, out_ref):
    hg = hg_ref[...]
    node = jnp.maximum(_md(hg, Wp_ref[...]) + bp_ref[...], 0.0)
    U = _md(node, W2p_ref[...]).reshape(3, SOLV_G, 32, D)
    v = _md(node, be2r_ref[...])
    ef = ef_ref[...]
    We1 = We1_ref[...]
    be1 = be1_ref[...]

    def msg(s, j):
        a = jnp.maximum(ef[:, j][:, None] * We1 + be1, 0.0)
        return jnp.sum(a[:, :, None] * U[s], axis=1) + v[s]

    bnn = bnn_ref[...]
    agg0 = msg(1, 0) + msg(2, 1) + msg(0, 3) + bnn
    agg1 = msg(0, 0) + msg(2, 2) + msg(1, 4) + bnn
    agg2 = msg(0, 1) + msg(1, 2) + msg(2, 5) + bnn
    mrel = jnp.maximum(jnp.stack([agg0, agg1, agg2]), 0.0)

    gi = _md(mrel, WihT_ref[...]) + bih_ref[...]
    gh = _md(node, WhhT_ref[...]) + bhh_ref[...]
    r = jax.nn.sigmoid(gi[..., :D] + gh[..., :D])
    z = jax.nn.sigmoid(gi[..., D:2 * D] + gh[..., D:2 * D])
    ng = jnp.tanh(gi[..., 2 * D:] + r * gh[..., 2 * D:])
    nod = (1.0 - z) * ng + z * node

    cat = jnp.concatenate([nod[0], nod[1], nod[2]], axis=1)
    o = jnp.maximum(
        lax.dot_general(cat, C1_ref[...], (((1,), (0,)), ((), ())),
                        preferred_element_type=jnp.float32) + C1b_ref[...], 0.0)
    o = jnp.maximum(
        lax.dot_general(o, C2_ref[...], (((1,), (0,)), ((), ())),
                        preferred_element_type=jnp.float32) + C2b_ref[...], 0.0)
    out_ref[...] = lax.dot_general(
        o, C3p_ref[...], (((1,), (0,)), ((), ())),
        preferred_element_type=jnp.float32) + C3b_ref[...]


def _solv_call(ef, hg, Wp, bp, We1, be1, W2p, be2r, bnn,
               WihT, WhhT, bih, bhh, C1, C1b, C2, C2b, C3p, C3b):
    g = B // SOLV_G

    def fixed(*shape):
        n = len(shape)
        return pl.BlockSpec(shape, lambda i, _n=n: (0,) * _n)

    return pl.pallas_call(
        _solv_body,
        grid=(g,),
        in_specs=[
            pl.BlockSpec((SOLV_G, 6), lambda i: (i, 0)),
            pl.BlockSpec((3, SOLV_G, D), lambda i: (0, i, 0)),
            fixed(D, D), fixed(1, D), fixed(1, 32), fixed(1, 32),
            fixed(D, 32 * D), fixed(D, D), fixed(1, D),
            fixed(D, 3 * D), fixed(D, 3 * D), fixed(1, 3 * D), fixed(1, 3 * D),
            fixed(3 * D, D), fixed(1, D), fixed(D, D), fixed(1, D),
            fixed(D, D), fixed(1, D),
        ],
        out_specs=pl.BlockSpec((SOLV_G, D), lambda i: (i, 0)),
        out_shape=jax.ShapeDtypeStruct((B, D), jnp.float32),
    )(ef, hg, Wp, bp, We1, be1, W2p, be2r, bnn,
      WihT, WhhT, bih, bhh, C1, C1b, C2, C2b, C3p, C3b)


def kernel(h1, h2, h3, solv1_x, solv2_x, inter_hb12, inter_hb13, inter_hb23,
           intra_hb1, intra_hb2, intra_hb3, W1, b1, W2, b2, Wp, bp,
           We1, be1, We2, be2, bnn, W_ih, W_hh, b_ih, b_hh,
           C1, C1b, C2, C2b, C3, C3b,
           edge_index1, edge_index2, edge_index3):
    ei = jnp.stack([edge_index1, edge_index2, edge_index3])
    zeros = jnp.zeros((APW,), jnp.float32)
    A = _sc_adj(ei, zeros).reshape(3, B, NP, NP)

    sol = jnp.stack([solv1_x, solv2_x, 1.0 - solv1_x - solv2_x],
                    axis=1)
    hg = _mol_call(sol, A,
                   h1.reshape(B, NP, D), h2.reshape(B, NP, D),
                   h3.reshape(B, NP, D),
                   W1, b1.reshape(1, D), W2, b2.reshape(1, D))

    ef = jnp.stack([inter_hb12, inter_hb13, inter_hb23,
                    intra_hb1, intra_hb2, intra_hb3], axis=1)
    W2p = We2.reshape(32, D, D).transpose(1, 0, 2).reshape(D, 32 * D)
    be2r = be2.reshape(D, D)
    C3p = jnp.pad(C3, ((0, 0), (0, D - NCOUT)))
    C3bp = jnp.pad(C3b, (0, D - NCOUT))

    out = _solv_call(ef, hg, Wp, bp.reshape(1, D), We1, be1.reshape(1, 32),
                     W2p, be2r, bnn.reshape(1, D),
                     W_ih.T, W_hh.T, b_ih.reshape(1, 3 * D),
                     b_hh.reshape(1, 3 * D),
                     C1, C1b.reshape(1, D), C2, C2b.reshape(1, D),
                     C3p, C3bp.reshape(1, D))
    return out[:, :NCOUT]


# speedup vs baseline: 14.9325x; 1.2492x over previous
import functools

import jax
import jax.numpy as jnp
from jax import lax
from jax.experimental import pallas as pl
from jax.experimental.pallas import tpu as pltpu
from jax.experimental.pallas import tpu_sc as plsc

B = 512
NP = 40
EP = 80
D = 128
NCOUT = 3
E = B * EP

NW = 32
GPW = B // NW
EPW = GPW * EP
APW = GPW * NP * NP

MOL_G = 16
SOLV_G = 64


@functools.partial(
    pl.kernel,
    out_type=jax.ShapeDtypeStruct((3, NW, APW), jnp.float32),
    mesh=plsc.VectorSubcoreMesh(core_axis_name="c", subcore_axis_name="s",
                                num_cores=2, num_subcores=16),
    compiler_params=pltpu.CompilerParams(needs_layout_passes=False),
    scratch_types=[
        pltpu.VMEM((EPW,), jnp.int32),
        pltpu.VMEM((EPW,), jnp.int32),
        pltpu.VMEM((APW,), jnp.float32),
    ],
)
def _sc_adj(ei_ref, zeros_ref, out_ref, src_v, dst_v, acc_v):
    wid = lax.axis_index("s") * 2 + lax.axis_index("c")
    base_e = wid * EPW
    ones = jnp.ones((16,), jnp.float32)
    for m in range(3):
        pltpu.sync_copy(ei_ref.at[m, 0, pl.ds(base_e, EPW)], src_v)
        pltpu.sync_copy(ei_ref.at[m, 1, pl.ds(base_e, EPW)], dst_v)
        pltpu.sync_copy(zeros_ref, acc_v)
        for i in range(EPW // 16):
            g_loc = i // 5
            s = src_v[pl.ds(i * 16, 16)]
            d = dst_v[pl.ds(i * 16, 16)]
            nb = (wid * GPW + g_loc) * NP
            flat = (d - nb) * NP + (s - nb) + g_loc * (NP * NP)
            plsc.addupdate_scatter(acc_v, [flat], ones)
        pltpu.sync_copy(acc_v, out_ref.at[m, wid])


FG = 64


def _m2(x, w):
    return lax.dot_general(x, w, (((1,), (0,)), ((), ())),
                           preferred_element_type=jnp.float32)


def _fused_body(sol_ref, ef_ref, A_ref, h1_ref, h2_ref, h3_ref,
                W1_ref, b1_ref, W2_ref, b2_ref,
                Wp_ref, bp_ref, We1_ref, be1_ref, W2p_ref, be2r_ref,
                exp_ref, bnn_ref, WihT_ref, WhhT_ref, bih_ref, bhh_ref,
                C1_ref, C1b_ref, C2_ref, C2b_ref, C3p_ref, C3b_ref,
                out_ref):
    sol = sol_ref[...]
    W1 = W1_ref[...]
    b1 = b1_ref[...]
    W2 = W2_ref[...]
    b2 = b2_ref[...]
    hgs = []
    for m, h_ref in enumerate((h1_ref, h2_ref, h3_ref)):
        A = A_ref[m]
        din = jnp.maximum(jnp.sum(A, axis=2), 1.0)
        dout = jnp.maximum(jnp.sum(A, axis=1), 1.0)
        Ah = A * lax.rsqrt(din)[:, :, None] * lax.rsqrt(dout)[:, None, :]
        t = h_ref[...]
        for W, bb in ((W1, b1), (W2, b2)):
            agg = lax.dot_general(
                Ah, t, (((2,), (1,)), ((0,), (0,))),
                preferred_element_type=jnp.float32)
            tw = _m2(agg.reshape(FG * NP, D), W)
            t = jnp.maximum(tw.reshape(FG, NP, D) + bb, 0.0)
        hgs.append(jnp.mean(t, axis=1) * sol[:, m][:, None])

    hg = jnp.concatenate(hgs, axis=0)
    node = jnp.maximum(_m2(hg, Wp_ref[...]) + bp_ref[...], 0.0)
    U = _m2(node, W2p_ref[...])
    v = _m2(node, be2r_ref[...])
    vsum = v[:FG] + v[FG:2 * FG] + v[2 * FG:] + bnn_ref[...]

    ef = ef_ref[...]
    We1 = We1_ref[...]
    be1 = be1_ref[...]
    expand = exp_ref[...]
    aexp = {}

    def get_aexp(j):
        if j not in aexp:
            acol = jnp.maximum(ef[:, j][:, None] * We1 + be1, 0.0)
            aexp[j] = _m2(acol, expand)
        return aexp[j]

    Ug = (U[:FG], U[FG:2 * FG], U[2 * FG:])
    combos = (((1, 0), (2, 1), (0, 3)),
              ((0, 0), (2, 2), (1, 4)),
              ((0, 1), (1, 2), (2, 5)))
    aggs = []
    for combo in combos:
        (s0, j0), (s1, j1), (s2, j2) = combo
        S = (get_aexp(j0) * Ug[s0] + get_aexp(j1) * Ug[s1]
             + get_aexp(j2) * Ug[s2])
        acc = vsum
        for k in range(32):
            acc = acc + S[:, k * D:(k + 1) * D]
        aggs.append(jnp.maximum(acc, 0.0))
    mrel = jnp.concatenate(aggs, axis=0)

    gi = _m2(mrel, WihT_ref[...]) + bih_ref[...]
    gh = _m2(node, WhhT_ref[...]) + bhh_ref[...]
    r = jax.nn.sigmoid(gi[:, :D] + gh[:, :D])
    z = jax.nn.sigmoid(gi[:, D:2 * D] + gh[:, D:2 * D])
    ng = jnp.tanh(gi[:, 2 * D:] + r * gh[:, 2 * D:])
    nod = (1.0 - z) * ng + z * node

    cat = jnp.concatenate([nod[:FG], nod[FG:2 * FG], nod[2 * FG:]], axis=1)
    o = jnp.maximum(_m2(cat, C1_ref[...]) + C1b_ref[...], 0.0)
    o = jnp.maximum(_m2(o, C2_ref[...]) + C2b_ref[...], 0.0)
    out_ref[...] = _m2(o, C3p_ref[...]) + C3b_ref[...]


def _fused_call(sol, ef, A, h1r, h2r, h3r, W1, b1, W2, b2,
                Wp, bp, We1, be1, W2p, be2r, expand, bnn,
                WihT, WhhT, bih, bhh, C1, C1b, C2, C2b, C3p, C3b):
    g = B // FG
    hspec = pl.BlockSpec((FG, NP, D), lambda i: (i, 0, 0))

    def fixed(*shape):
        n = len(shape)
        return pl.BlockSpec(shape, lambda i, _n=n: (0,) * _n)

    return pl.pallas_call(
        _fused_body,
        grid=(g,),
        in_specs=[
            pl.BlockSpec((FG, 3), lambda i: (i, 0)),
            pl.BlockSpec((FG, 6), lambda i: (i, 0)),
            pl.BlockSpec((3, FG, NP, NP), lambda i: (0, i, 0, 0)),
            hspec, hspec, hspec,
            fixed(D, D), fixed(1, D), fixed(D, D), fixed(1, D),
            fixed(D, D), fixed(1, D), fixed(1, 32), fixed(1, 32),
            fixed(D, 32 * D), fixed(D, D), fixed(32, 32 * D), fixed(1, D),
            fixed(D, 3 * D), fixed(D, 3 * D), fixed(1, 3 * D), fixed(1, 3 * D),
            fixed(3 * D, D), fixed(1, D), fixed(D, D), fixed(1, D),
            fixed(D, D), fixed(1, D),
        ],
        out_specs=pl.BlockSpec((FG, D), lambda i: (i, 0)),
        out_shape=jax.ShapeDtypeStruct((B, D), jnp.float32),
    )(sol, ef, A, h1r, h2r, h3r, W1, b1, W2, b2,
      Wp, bp, We1, be1, W2p, be2r, expand, bnn,
      WihT, WhhT, bih, bhh, C1, C1b, C2, C2b, C3p, C3b)


def _post(A, h1, h2, h3, solv1_x, solv2_x,
          inter_hb12, inter_hb13, inter_hb23, intra_hb1, intra_hb2, intra_hb3,
          W1, b1, W2, b2, Wp, bp, We1, be1, We2, be2, bnn,
          W_ih, W_hh, b_ih, b_hh, C1, C1b, C2, C2b, C3, C3b):
    sol = jnp.stack([solv1_x, solv2_x, 1.0 - solv1_x - solv2_x], axis=1)
    ef = jnp.stack([inter_hb12, inter_hb13, inter_hb23,
                    intra_hb1, intra_hb2, intra_hb3], axis=1)
    W2p = We2.reshape(32, D, D).transpose(1, 0, 2).reshape(D, 32 * D)
    be2r = be2.reshape(D, D)
    expand = jnp.repeat(jnp.eye(32, dtype=jnp.float32), D, axis=1)
    C3p = jnp.pad(C3, ((0, 0), (0, D - NCOUT)))
    C3bp = jnp.pad(C3b, (0, D - NCOUT))
    out = _fused_call(sol, ef, A,
                      h1.reshape(B, NP, D), h2.reshape(B, NP, D),
                      h3.reshape(B, NP, D),
                      W1, b1.reshape(1, D), W2, b2.reshape(1, D),
                      Wp, bp.reshape(1, D), We1, be1.reshape(1, 32),
                      W2p, be2r, expand, bnn.reshape(1, D),
                      W_ih.T, W_hh.T, b_ih.reshape(1, 3 * D),
                      b_hh.reshape(1, 3 * D),
                      C1, C1b.reshape(1, D), C2, C2b.reshape(1, D),
                      C3p, C3bp.reshape(1, D))
    return out[:, :NCOUT]


def kernel(h1, h2, h3, solv1_x, solv2_x, inter_hb12, inter_hb13, inter_hb23,
           intra_hb1, intra_hb2, intra_hb3, W1, b1, W2, b2, Wp, bp,
           We1, be1, We2, be2, bnn, W_ih, W_hh, b_ih, b_hh,
           C1, C1b, C2, C2b, C3, C3b,
           edge_index1, edge_index2, edge_index3):
    ei = jnp.stack([edge_index1, edge_index2, edge_index3])
    zeros = jnp.zeros((APW,), jnp.float32)
    A = _sc_adj(ei, zeros).reshape(3, B, NP, NP)
    return _post(A, h1, h2, h3, solv1_x, solv2_x,
                 inter_hb12, inter_hb13, inter_hb23,
                 intra_hb1, intra_hb2, intra_hb3,
                 W1, b1, W2, b2, Wp, bp, We1, be1, We2, be2, bnn,
                 W_ih, W_hh, b_ih, b_hh, C1, C1b, C2, C2b, C3, C3b)


# speedup vs baseline: 17.9420x; 1.2015x over previous
import functools

import jax
import jax.numpy as jnp
from jax import lax
from jax.experimental import pallas as pl
from jax.experimental.pallas import tpu as pltpu
from jax.experimental.pallas import tpu_sc as plsc

B = 512
NP = 40
EP = 80
D = 128
NCOUT = 3
E = B * EP

NW = 32
GPW = B // NW
EPW = GPW * EP
APW = GPW * NP * NP

MOL_G = 16
SOLV_G = 64


RPW = GPW * NP
CW = 48


@functools.partial(
    pl.kernel,
    out_type=jax.ShapeDtypeStruct((3, B * NP, 128), jnp.float32),
    mesh=plsc.VectorSubcoreMesh(core_axis_name="c", subcore_axis_name="s",
                                num_cores=2, num_subcores=16),
    compiler_params=pltpu.CompilerParams(needs_layout_passes=False),
    scratch_types=[
        pltpu.VMEM((EPW,), jnp.int32),
        pltpu.VMEM((EPW,), jnp.int32),
        pltpu.VMEM((RPW, 128), jnp.float32),
    ],
)
def _sc_adj(ei_ref, out_ref, src_v, dst_v, acc_v):
    wid = lax.axis_index("s") * 2 + lax.axis_index("c")
    base_e = wid * EPW
    ones = jnp.ones((16,), jnp.float32)
    zv = jnp.zeros((16,), jnp.float32)

    def _zero_rows(i, carry):
        r0 = i * 8
        for rr in range(8):
            for c in range(CW // 16):
                acc_v[r0 + rr, pl.ds(c * 16, 16)] = zv
        return carry

    for m in range(3):
        pltpu.sync_copy(ei_ref.at[m, 0, pl.ds(base_e, EPW)], src_v)
        pltpu.sync_copy(ei_ref.at[m, 1, pl.ds(base_e, EPW)], dst_v)
        lax.fori_loop(0, RPW // 8, _zero_rows, 0)
        for i in range(EPW // 16):
            g_loc = i // 5
            s = src_v[pl.ds(i * 16, 16)]
            d = dst_v[pl.ds(i * 16, 16)]
            nb = (wid * GPW + g_loc) * NP
            row = (d - nb) + g_loc * NP
            plsc.addupdate_scatter(acc_v, [row, s - nb], ones)
        pltpu.sync_copy(acc_v, out_ref.at[m, pl.ds(wid * RPW, RPW)])


FG = 64


def _m2(x, w):
    return lax.dot_general(x, w, (((1,), (0,)), ((), ())),
                           preferred_element_type=jnp.float32)


def _m2t(x, w):
    return lax.dot_general(x, w, (((1,), (1,)), ((), ())),
                           preferred_element_type=jnp.float32)


def _bdot(Ah, t):
    return lax.dot_general(Ah, t, (((2,), (1,)), ((0,), (0,))),
                           preferred_element_type=jnp.float32)


def _fused_body(sol_ref, ef_ref, A_ref, xw1_ref, xw2_ref, xw3_ref,
                b1_ref, W2_ref, b2_ref,
                Wp_ref, bp_ref, We1_ref, be1_ref, We2r_ref, be2r_ref,
                exp_ref, bnn_ref, Wih_ref, Whh_ref, bih_ref, bhh_ref,
                C1_ref, C1b_ref, C2_ref, C2b_ref, C3_ref, C3b_ref,
                out_ref):
    sol = sol_ref[...]
    b1 = b1_ref[...]
    W2 = W2_ref[...]
    b2 = b2_ref[...]
    A3 = A_ref[...].reshape(3, FG, NP, 128)[:, :, :, :NP]
    hgs = []
    for m, xw_ref in enumerate((xw1_ref, xw2_ref, xw3_ref)):
        A = A3[m]
        din = jnp.maximum(jnp.sum(A, axis=2), 1.0)
        dout = jnp.maximum(jnp.sum(A, axis=1), 1.0)
        Ah = A * lax.rsqrt(din)[:, :, None] * lax.rsqrt(dout)[:, None, :]
        xw = xw_ref[...].reshape(FG, NP, D)
        t = jnp.maximum(_bdot(Ah, xw) + b1, 0.0)
        tw = _m2(_bdot(Ah, t).reshape(FG * NP, D), W2)
        t = jnp.maximum(tw.reshape(FG, NP, D) + b2, 0.0)
        hgs.append(jnp.mean(t, axis=1) * sol[:, m][:, None])

    hg = jnp.concatenate(hgs, axis=0)
    node = jnp.maximum(_m2(hg, Wp_ref[...]) + bp_ref[...], 0.0)
    Uk = [_m2(node, We2r_ref[k]) for k in range(32)]
    v = _m2(node, be2r_ref[...])
    vsum = v[:FG] + v[FG:2 * FG] + v[2 * FG:] + bnn_ref[...]

    ef = ef_ref[...]
    We1 = We1_ref[...]
    be1 = be1_ref[...]
    expand = exp_ref[...]
    aexp = {}

    def get_aexp(j):
        if j not in aexp:
            acol = jnp.maximum(ef[:, j][:, None] * We1 + be1, 0.0)
            aexp[j] = _m2(acol, expand)
        return aexp[j]

    combos = (((1, 0), (2, 1), (0, 3)),
              ((0, 0), (2, 2), (1, 4)),
              ((0, 1), (1, 2), (2, 5)))
    aggs = []
    for combo in combos:
        acc = vsum
        for k in range(32):
            ks = slice(k * D, (k + 1) * D)
            for sm, j in combo:
                acc = acc + get_aexp(j)[:, ks] * Uk[k][sm * FG:(sm + 1) * FG]
        aggs.append(jnp.maximum(acc, 0.0))
    mrel = jnp.concatenate(aggs, axis=0)

    gi = _m2t(mrel, Wih_ref[...]) + bih_ref[...]
    gh = _m2t(node, Whh_ref[...]) + bhh_ref[...]
    r = jax.nn.sigmoid(gi[:, :D] + gh[:, :D])
    z = jax.nn.sigmoid(gi[:, D:2 * D] + gh[:, D:2 * D])
    ng = jnp.tanh(gi[:, 2 * D:] + r * gh[:, 2 * D:])
    nod = (1.0 - z) * ng + z * node

    cat = jnp.concatenate([nod[:FG], nod[FG:2 * FG], nod[2 * FG:]], axis=1)
    o = jnp.maximum(_m2(cat, C1_ref[...]) + C1b_ref[...], 0.0)
    o = jnp.maximum(_m2(o, C2_ref[...]) + C2b_ref[...], 0.0)
    out_ref[...] = _m2(o, C3_ref[...]) + C3b_ref[...]


def _xw_body(h1_ref, h2_ref, h3_ref, W1_ref, o1_ref, o2_ref, o3_ref):
    W1 = W1_ref[...]
    o1_ref[...] = _m2(h1_ref[...], W1)
    o2_ref[...] = _m2(h2_ref[...], W1)
    o3_ref[...] = _m2(h3_ref[...], W1)


def _xw_call(h1, h2, h3, W1):
    g = B // FG
    rows = FG * NP
    rspec = pl.BlockSpec((rows, D), lambda i: (i, 0))
    sds = jax.ShapeDtypeStruct((B * NP, D), jnp.float32)
    return pl.pallas_call(
        _xw_body,
        grid=(g,),
        in_specs=[rspec, rspec, rspec,
                  pl.BlockSpec((D, D), lambda i: (0, 0))],
        out_specs=[rspec, rspec, rspec],
        out_shape=[sds, sds, sds],
    )(h1, h2, h3, W1)


def _fused_call(sol, ef, A, xw1, xw2, xw3, b1, W2, b2,
                Wp, bp, We1, be1, We2r, be2r, expand, bnn,
                Wih, Whh, bih, bhh, C1, C1b, C2, C2b, C3, C3b):
    g = B // FG
    rspec = pl.BlockSpec((FG * NP, D), lambda i: (i, 0))

    def fixed(*shape):
        n = len(shape)
        return pl.BlockSpec(shape, lambda i, _n=n: (0,) * _n)

    return pl.pallas_call(
        _fused_body,
        grid=(g,),
        in_specs=[
            pl.BlockSpec((FG, 3), lambda i: (i, 0)),
            pl.BlockSpec((FG, 6), lambda i: (i, 0)),
            pl.BlockSpec((3, FG * NP, 128), lambda i: (0, i, 0)),
            rspec, rspec, rspec,
            fixed(1, D), fixed(D, D), fixed(1, D),
            fixed(D, D), fixed(1, D), fixed(1, 32), fixed(1, 32),
            fixed(32, D, D), fixed(D, D), fixed(32, 32 * D), fixed(1, D),
            fixed(3 * D, D), fixed(3 * D, D), fixed(1, 3 * D), fixed(1, 3 * D),
            fixed(3 * D, D), fixed(1, D), fixed(D, D), fixed(1, D),
            fixed(D, NCOUT), fixed(1, NCOUT),
        ],
        out_specs=pl.BlockSpec((FG, NCOUT), lambda i: (i, 0)),
        out_shape=jax.ShapeDtypeStruct((B, NCOUT), jnp.float32),
    )(sol, ef, A, xw1, xw2, xw3, b1, W2, b2,
      Wp, bp, We1, be1, We2r, be2r, expand, bnn,
      Wih, Whh, bih, bhh, C1, C1b, C2, C2b, C3, C3b)


def _post(A, h1, h2, h3, solv1_x, solv2_x,
          inter_hb12, inter_hb13, inter_hb23, intra_hb1, intra_hb2, intra_hb3,
          W1, b1, W2, b2, Wp, bp, We1, be1, We2, be2, bnn,
          W_ih, W_hh, b_ih, b_hh, C1, C1b, C2, C2b, C3, C3b):
    sol = jnp.stack([solv1_x, solv2_x, 1.0 - solv1_x - solv2_x], axis=1)
    ef = jnp.stack([inter_hb12, inter_hb13, inter_hb23,
                    intra_hb1, intra_hb2, intra_hb3], axis=1)
    We2r = We2.reshape(32, D, D)
    be2r = be2.reshape(D, D)
    expand = jnp.repeat(jnp.eye(32, dtype=jnp.float32), D, axis=1)
    xw1, xw2, xw3 = _xw_call(h1, h2, h3, W1)
    return _fused_call(sol, ef, A, xw1, xw2, xw3,
                       b1.reshape(1, D), W2, b2.reshape(1, D),
                       Wp, bp.reshape(1, D), We1, be1.reshape(1, 32),
                       We2r, be2r, expand, bnn.reshape(1, D),
                       W_ih, W_hh, b_ih.reshape(1, 3 * D),
                       b_hh.reshape(1, 3 * D),
                       C1, C1b.reshape(1, D), C2, C2b.reshape(1, D),
                       C3, C3b.reshape(1, NCOUT))


def kernel(h1, h2, h3, solv1_x, solv2_x, inter_hb12, inter_hb13, inter_hb23,
           intra_hb1, intra_hb2, intra_hb3, W1, b1, W2, b2, Wp, bp,
           We1, be1, We2, be2, bnn, W_ih, W_hh, b_ih, b_hh,
           C1, C1b, C2, C2b, C3, C3b,
           edge_index1, edge_index2, edge_index3):
    ei = jnp.stack([edge_index1, edge_index2, edge_index3])
    A = _sc_adj(ei)
    return _post(A, h1, h2, h3, solv1_x, solv2_x,
                 inter_hb12, inter_hb13, inter_hb23,
                 intra_hb1, intra_hb2, intra_hb3,
                 W1, b1, W2, b2, Wp, bp, We1, be1, We2, be2, bnn,
                 W_ih, W_hh, b_ih, b_hh, C1, C1b, C2, C2b, C3, C3b)


# speedup vs baseline: 19.8609x; 1.1070x over previous
import functools

import numpy as np

import jax
import jax.numpy as jnp
from jax import lax
from jax.experimental import pallas as pl
from jax.experimental.pallas import tpu as pltpu
from jax.experimental.pallas import tpu_sc as plsc

B = 512
NP = 40
EP = 80
D = 128
NCOUT = 3
E = B * EP

NW = 32
GPW = B // NW
EPW = GPW * EP
APW = GPW * NP * NP

MOL_G = 16
SOLV_G = 64


RPW = GPW * NP
CW = 48


@functools.partial(
    pl.kernel,
    out_type=jax.ShapeDtypeStruct((3, B * NP, 128), jnp.float32),
    mesh=plsc.VectorSubcoreMesh(core_axis_name="c", subcore_axis_name="s",
                                num_cores=2, num_subcores=16),
    compiler_params=pltpu.CompilerParams(needs_layout_passes=False),
    scratch_types=[
        pltpu.VMEM((EPW,), jnp.int32),
        pltpu.VMEM((EPW,), jnp.int32),
        pltpu.VMEM((RPW, 128), jnp.float32),
    ],
)
def _sc_adj(ei_ref, out_ref, src_v, dst_v, acc_v):
    wid = lax.axis_index("s") * 2 + lax.axis_index("c")
    base_e = wid * EPW
    ones = jnp.ones((16,), jnp.float32)
    zv = jnp.zeros((16,), jnp.float32)

    def _zero_rows(i, carry):
        r0 = i * 8
        for rr in range(8):
            for c in range(CW // 16):
                acc_v[r0 + rr, pl.ds(c * 16, 16)] = zv
        return carry

    for m in range(3):
        pltpu.sync_copy(ei_ref.at[m, 0, pl.ds(base_e, EPW)], src_v)
        pltpu.sync_copy(ei_ref.at[m, 1, pl.ds(base_e, EPW)], dst_v)
        lax.fori_loop(0, RPW // 8, _zero_rows, 0)
        for i in range(EPW // 16):
            g_loc = i // 5
            s = src_v[pl.ds(i * 16, 16)]
            d = dst_v[pl.ds(i * 16, 16)]
            nb = (wid * GPW + g_loc) * NP
            row = (d - nb) + g_loc * NP
            plsc.addupdate_scatter(acc_v, [row, s - nb], ones)
        pltpu.sync_copy(acc_v, out_ref.at[m, pl.ds(wid * RPW, RPW)])


FG = 64

_EXPAND = np.repeat(np.eye(32, dtype=np.float32), D, axis=1)


def _m2(x, w):
    return lax.dot_general(x, w, (((1,), (0,)), ((), ())),
                           preferred_element_type=jnp.float32)


def _m2t(x, w):
    return lax.dot_general(x, w, (((1,), (1,)), ((), ())),
                           preferred_element_type=jnp.float32)


def _bdot(Ah, t):
    return lax.dot_general(Ah, t, (((2,), (1,)), ((0,), (0,))),
                           preferred_element_type=jnp.float32)


def _fused_body(sol_ref, ef_ref, A_ref, h1_ref, h2_ref, h3_ref,
                W1_ref, b1_ref, W2_ref, b2_ref,
                Wp_ref, bp_ref, We1_ref, be1_ref, We2r_ref, be2r_ref,
                exp_ref, bnn_ref, Wih_ref, Whh_ref, bih_ref, bhh_ref,
                C1_ref, C1b_ref, C2_ref, C2b_ref, C3_ref, C3b_ref,
                out_ref):
    sol = sol_ref[...]
    W1 = W1_ref[...]
    b1 = b1_ref[...]
    W2 = W2_ref[...]
    b2 = b2_ref[...]
    A3 = A_ref[...].reshape(3, FG, NP, 128)[:, :, :, :NP]
    hgs = []
    for m, h_ref in enumerate((h1_ref, h2_ref, h3_ref)):
        A = A3[m]
        din = jnp.maximum(jnp.sum(A, axis=2), 1.0)
        dout = jnp.maximum(jnp.sum(A, axis=1), 1.0)
        Ah = A * lax.rsqrt(din)[:, :, None] * lax.rsqrt(dout)[:, None, :]
        xw = _m2(h_ref[...], W1).reshape(FG, NP, D)
        t = jnp.maximum(_bdot(Ah, xw) + b1, 0.0)
        tw = _m2(_bdot(Ah, t).reshape(FG * NP, D), W2)
        t = jnp.maximum(tw.reshape(FG, NP, D) + b2, 0.0)
        hgs.append(jnp.mean(t, axis=1) * sol[:, m][:, None])

    hg = jnp.concatenate(hgs, axis=0)
    node = jnp.maximum(_m2(hg, Wp_ref[...]) + bp_ref[...], 0.0)
    Uk = [_m2(node, We2r_ref[k]) for k in range(32)]
    v = _m2(node, be2r_ref[...])
    vsum = v[:FG] + v[FG:2 * FG] + v[2 * FG:] + bnn_ref[...]

    ef = ef_ref[...]
    We1 = We1_ref[...]
    be1 = be1_ref[...]
    expand = exp_ref[...]
    aexp = {}

    def get_aexp(j):
        if j not in aexp:
            acol = jnp.maximum(ef[:, j][:, None] * We1 + be1, 0.0)
            aexp[j] = _m2(acol, expand)
        return aexp[j]

    combos = (((1, 0), (2, 1), (0, 3)),
              ((0, 0), (2, 2), (1, 4)),
              ((0, 1), (1, 2), (2, 5)))
    aggs = []
    for combo in combos:
        acc = vsum
        for k in range(32):
            ks = slice(k * D, (k + 1) * D)
            for sm, j in combo:
                acc = acc + get_aexp(j)[:, ks] * Uk[k][sm * FG:(sm + 1) * FG]
        aggs.append(jnp.maximum(acc, 0.0))
    mrel = jnp.concatenate(aggs, axis=0)

    gi = _m2t(mrel, Wih_ref[...]) + bih_ref[...]
    gh = _m2t(node, Whh_ref[...]) + bhh_ref[...]
    r = jax.nn.sigmoid(gi[:, :D] + gh[:, :D])
    z = jax.nn.sigmoid(gi[:, D:2 * D] + gh[:, D:2 * D])
    ng = jnp.tanh(gi[:, 2 * D:] + r * gh[:, 2 * D:])
    nod = (1.0 - z) * ng + z * node

    cat = jnp.concatenate([nod[:FG], nod[FG:2 * FG], nod[2 * FG:]], axis=1)
    o = jnp.maximum(_m2(cat, C1_ref[...]) + C1b_ref[...], 0.0)
    o = jnp.maximum(_m2(o, C2_ref[...]) + C2b_ref[...], 0.0)
    out_ref[...] = _m2(o, C3_ref[...]) + C3b_ref[...]


def _fused_call(sol, ef, A, h1, h2, h3, W1, b1, W2, b2,
                Wp, bp, We1, be1, We2r, be2r, expand, bnn,
                Wih, Whh, bih, bhh, C1, C1b, C2, C2b, C3, C3b):
    g = B // FG
    rspec = pl.BlockSpec((FG * NP, D), lambda i: (i, 0))

    def fixed(*shape):
        n = len(shape)
        return pl.BlockSpec(shape, lambda i, _n=n: (0,) * _n)

    return pl.pallas_call(
        _fused_body,
        grid=(g,),
        in_specs=[
            pl.BlockSpec((FG, 3), lambda i: (i, 0)),
            pl.BlockSpec((FG, 6), lambda i: (i, 0)),
            pl.BlockSpec((3, FG * NP, 128), lambda i: (0, i, 0)),
            rspec, rspec, rspec,
            fixed(D, D), fixed(1, D), fixed(D, D), fixed(1, D),
            fixed(D, D), fixed(1, D), fixed(1, 32), fixed(1, 32),
            fixed(32, D, D), fixed(D, D), fixed(32, 32 * D), fixed(1, D),
            fixed(3 * D, D), fixed(3 * D, D), fixed(1, 3 * D), fixed(1, 3 * D),
            fixed(3 * D, D), fixed(1, D), fixed(D, D), fixed(1, D),
            fixed(D, NCOUT), fixed(1, NCOUT),
        ],
        out_specs=pl.BlockSpec((FG, NCOUT), lambda i: (i, 0)),
        out_shape=jax.ShapeDtypeStruct((B, NCOUT), jnp.float32),
    )(sol, ef, A, h1, h2, h3, W1, b1, W2, b2,
      Wp, bp, We1, be1, We2r, be2r, expand, bnn,
      Wih, Whh, bih, bhh, C1, C1b, C2, C2b, C3, C3b)


def _post(A, h1, h2, h3, solv1_x, solv2_x,
          inter_hb12, inter_hb13, inter_hb23, intra_hb1, intra_hb2, intra_hb3,
          W1, b1, W2, b2, Wp, bp, We1, be1, We2, be2, bnn,
          W_ih, W_hh, b_ih, b_hh, C1, C1b, C2, C2b, C3, C3b):
    sol = jnp.stack([solv1_x, solv2_x, 1.0 - solv1_x - solv2_x], axis=1)
    ef = jnp.stack([inter_hb12, inter_hb13, inter_hb23,
                    intra_hb1, intra_hb2, intra_hb3], axis=1)
    We2r = We2.reshape(32, D, D)
    be2r = be2.reshape(D, D)
    expand = jnp.asarray(_EXPAND)
    return _fused_call(sol, ef, A, h1, h2, h3,
                       W1, b1.reshape(1, D), W2, b2.reshape(1, D),
                       Wp, bp.reshape(1, D), We1, be1.reshape(1, 32),
                       We2r, be2r, expand, bnn.reshape(1, D),
                       W_ih, W_hh, b_ih.reshape(1, 3 * D),
                       b_hh.reshape(1, 3 * D),
                       C1, C1b.reshape(1, D), C2, C2b.reshape(1, D),
                       C3, C3b.reshape(1, NCOUT))


def kernel(h1, h2, h3, solv1_x, solv2_x, inter_hb12, inter_hb13, inter_hb23,
           intra_hb1, intra_hb2, intra_hb3, W1, b1, W2, b2, Wp, bp,
           We1, be1, We2, be2, bnn, W_ih, W_hh, b_ih, b_hh,
           C1, C1b, C2, C2b, C3, C3b,
           edge_index1, edge_index2, edge_index3):
    ei = jnp.stack([edge_index1, edge_index2, edge_index3])
    A = _sc_adj(ei)
    return _post(A, h1, h2, h3, solv1_x, solv2_x,
                 inter_hb12, inter_hb13, inter_hb23,
                 intra_hb1, intra_hb2, intra_hb3,
                 W1, b1, W2, b2, Wp, bp, We1, be1, We2, be2, bnn,
                 W_ih, W_hh, b_ih, b_hh, C1, C1b, C2, C2b, C3, C3b)
